# baseline (speedup 1.0000x reference)
import functools

import jax
import jax.numpy as jnp
from jax.experimental import pallas as pl
from jax.experimental.pallas import tpu as pltpu

_B, _H, _W, _C = 16, 512, 512, 3
_P = 8
_OUT = 64
_BG = 8
_WIN_Y = 136
_WIN_X = 256


def _weights_un(length, off, win, *, transposed):
    lf = length.astype(jnp.float32)
    inv_scale = lf * (1.0 / _OUT)
    ks = jnp.maximum(inv_scale, 1.0)
    if transposed:
        shape = (_OUT, win)
        s_dim, i_dim = 0, 1
    else:
        shape = (win, _OUT)
        s_dim, i_dim = 1, 0
    s = jax.lax.broadcasted_iota(jnp.int32, shape, s_dim)
    i = jax.lax.broadcasted_iota(jnp.int32, shape, i_dim)
    sample = (s.astype(jnp.float32) + 0.5) * inv_scale - 0.5
    r = i.astype(jnp.float32) - off.astype(jnp.float32)
    x = jnp.abs(sample - r) * (1.0 / ks)
    w = jnp.maximum(0.0, 1.0 - x)
    return w * ((r >= 0.0) & (r < lf)).astype(jnp.float32)


def _dot(a, b):
    return jax.lax.dot_general(
        a, b, (((1,), (0,)), ((), ())),
        precision=jax.lax.Precision.DEFAULT,
        preferred_element_type=jnp.float32)


def _body(g_ref, boxes_ref, pg_ref, out_ref):
    gb = pl.program_id(0)

    @pl.when(gb == 0)
    def _init():
        out_ref[...] = jnp.zeros_like(out_ref)

    ones_x = jnp.ones((1, _WIN_X), jnp.float32)
    accs = [jnp.zeros((_OUT, _OUT), jnp.float32) for _ in range(_C)]
    for bi in range(_BG):
        b = gb * _BG + bi
        for p in range(_P):
            ymin = boxes_ref[b, p, 0]
            xmin = boxes_ref[b, p, 1]
            ph = boxes_ref[b, p, 2]
            pw = boxes_ref[b, p, 3]
            ys = (jnp.minimum(ymin, _H - _WIN_Y + 5) // 8) * 8
            xs = (jnp.minimum(xmin, _W - _WIN_X + 1) // 128) * 128
            wyt = _weights_un(ph, ymin - ys, _WIN_Y, transposed=True)
            wx = _weights_un(pw, xmin - xs, _WIN_X, transposed=False)
            rty = 1.0 / jnp.sum(wyt, axis=1, keepdims=True)
            rtx = 1.0 / _dot(ones_x, wx)
            scale = rty * rtx
            ms = []
            for c in range(_C):
                crop = g_ref[bi, c, pl.ds(ys, _WIN_Y), pl.ds(xs, _WIN_X)]
                ms.append(_dot(wyt, crop))
            m_all = jnp.concatenate(ms, axis=0)
            o_all = _dot(m_all, wx)
            for c in range(_C):
                o = o_all[c * _OUT:(c + 1) * _OUT, :]
                accs[c] = accs[c] + o * scale
    for c in range(_C):
        out_ref[c, :, :] += accs[c]

    @pl.when(gb == (_B // _BG) - 1)
    def _finish():
        out_ref[...] = out_ref[...] * pg_ref[...]


@functools.partial(jax.jit, static_argnames=())
def kernel(gradients, patch_boxes, transform_decisions, patch_grads):
    del transform_decisions
    g = jnp.transpose(gradients, (0, 3, 1, 2))
    pg = jnp.transpose(patch_grads, (2, 0, 1))
    out = pl.pallas_call(
        _body,
        grid=(_B // _BG,),
        in_specs=[
            pl.BlockSpec((_BG, _C, _H, _W), lambda i: (i, 0, 0, 0)),
            pl.BlockSpec(memory_space=pltpu.SMEM),
            pl.BlockSpec((_C, _OUT, _OUT), lambda i: (0, 0, 0)),
        ],
        out_specs=pl.BlockSpec((_C, _OUT, _OUT), lambda i: (0, 0, 0)),
        out_shape=jax.ShapeDtypeStruct((_C, _OUT, _OUT), jnp.float32),
    )(g, patch_boxes, pg)
    return jnp.transpose(out, (1, 2, 0))


# speedup vs baseline: 1.0827x; 1.0827x over previous
import functools

import jax
import jax.numpy as jnp
from jax.experimental import pallas as pl
from jax.experimental.pallas import tpu as pltpu

_B, _H, _W, _C = 16, 512, 512, 3
_P = 8
_OUT = 64
_BG = 2
_WIN_Y = 136
_WIN_X = 256


def _weights_un(length, off, win, *, transposed):
    lf = length.astype(jnp.float32)
    inv_scale = lf * (1.0 / _OUT)
    ks = jnp.maximum(inv_scale, 1.0)
    if transposed:
        shape = (_OUT, win)
        s_dim, i_dim = 0, 1
    else:
        shape = (win, _OUT)
        s_dim, i_dim = 1, 0
    s = jax.lax.broadcasted_iota(jnp.int32, shape, s_dim)
    i = jax.lax.broadcasted_iota(jnp.int32, shape, i_dim)
    sample = (s.astype(jnp.float32) + 0.5) * inv_scale - 0.5
    r = i.astype(jnp.float32) - off.astype(jnp.float32)
    x = jnp.abs(sample - r) * (1.0 / ks)
    w = jnp.maximum(0.0, 1.0 - x)
    return w * ((r >= 0.0) & (r < lf)).astype(jnp.float32)


def _dot(a, b):
    return jax.lax.dot_general(
        a, b, (((1,), (0,)), ((), ())),
        precision=jax.lax.Precision.DEFAULT,
        preferred_element_type=jnp.float32)


def _body(g_ref, boxes_ref, pg_ref, out_ref):
    gb = pl.program_id(0)

    @pl.when(gb == 0)
    def _init():
        out_ref[...] = jnp.zeros_like(out_ref)

    ones_x = jnp.ones((1, _WIN_X), jnp.float32)
    accs = [jnp.zeros((_OUT, _OUT), jnp.float32) for _ in range(_C)]
    for bi in range(_BG):
        b = gb * _BG + bi
        for p in range(_P):
            ymin = boxes_ref[b, p, 0]
            xmin = boxes_ref[b, p, 1]
            ph = boxes_ref[b, p, 2]
            pw = boxes_ref[b, p, 3]
            ys = (jnp.minimum(ymin, _H - _WIN_Y + 5) // 8) * 8
            xs = (jnp.minimum(xmin, _W - _WIN_X + 1) // 128) * 128
            wyt = _weights_un(ph, ymin - ys, _WIN_Y, transposed=True)
            wx = _weights_un(pw, xmin - xs, _WIN_X, transposed=False)
            rty = 1.0 / jnp.sum(wyt, axis=1, keepdims=True)
            rtx = 1.0 / _dot(ones_x, wx)
            scale = rty * rtx
            ms = []
            for c in range(_C):
                crop = g_ref[bi, c, pl.ds(ys, _WIN_Y), pl.ds(xs, _WIN_X)]
                ms.append(_dot(wyt, crop))
            m_all = jnp.concatenate(ms, axis=0)
            o_all = _dot(m_all, wx)
            for c in range(_C):
                o = o_all[c * _OUT:(c + 1) * _OUT, :]
                accs[c] = accs[c] + o * scale
    for c in range(_C):
        out_ref[c, :, :] += accs[c]

    @pl.when(gb == (_B // _BG) - 1)
    def _finish():
        out_ref[...] = out_ref[...] * pg_ref[...]


@functools.partial(jax.jit, static_argnames=())
def kernel(gradients, patch_boxes, transform_decisions, patch_grads):
    del transform_decisions
    g = jnp.transpose(gradients, (0, 3, 1, 2))
    pg = jnp.transpose(patch_grads, (2, 0, 1))
    out = pl.pallas_call(
        _body,
        grid=(_B // _BG,),
        in_specs=[
            pl.BlockSpec((_BG, _C, _H, _W), lambda i: (i, 0, 0, 0)),
            pl.BlockSpec(memory_space=pltpu.SMEM),
            pl.BlockSpec((_C, _OUT, _OUT), lambda i: (0, 0, 0)),
        ],
        out_specs=pl.BlockSpec((_C, _OUT, _OUT), lambda i: (0, 0, 0)),
        out_shape=jax.ShapeDtypeStruct((_C, _OUT, _OUT), jnp.float32),
    )(g, patch_boxes, pg)
    return jnp.transpose(out, (1, 2, 0))


# speedup vs baseline: 1.1100x; 1.0252x over previous
import functools

import jax
import jax.numpy as jnp
from jax.experimental import pallas as pl
from jax.experimental.pallas import tpu as pltpu

_B, _H, _W, _C = 16, 512, 512, 3
_P = 8
_OUT = 64
_BG = 4
_WIN_Y = 136
_WIN_X = 256


def _weights_un(length, off, win, *, transposed):
    lf = length.astype(jnp.float32)
    inv_scale = lf * (1.0 / _OUT)
    ks = jnp.maximum(inv_scale, 1.0)
    if transposed:
        shape = (_OUT, win)
        s_dim, i_dim = 0, 1
    else:
        shape = (win, _OUT)
        s_dim, i_dim = 1, 0
    s = jax.lax.broadcasted_iota(jnp.int32, shape, s_dim)
    i = jax.lax.broadcasted_iota(jnp.int32, shape, i_dim)
    sample = (s.astype(jnp.float32) + 0.5) * inv_scale - 0.5
    r = i.astype(jnp.float32) - off.astype(jnp.float32)
    x = jnp.abs(sample - r) * (1.0 / ks)
    w = jnp.maximum(0.0, 1.0 - x)
    return w * ((r >= 0.0) & (r < lf)).astype(jnp.float32)


def _dot(a, b):
    return jax.lax.dot_general(
        a, b, (((1,), (0,)), ((), ())),
        precision=jax.lax.Precision.DEFAULT,
        preferred_element_type=jnp.float32)


def _body(g_ref, boxes_ref, pg_ref, out_ref):
    gb = pl.program_id(0)

    @pl.when(gb == 0)
    def _init():
        out_ref[...] = jnp.zeros_like(out_ref)

    ones_x = jnp.ones((1, _WIN_X), jnp.float32)
    accs = [jnp.zeros((_OUT, _OUT), jnp.float32) for _ in range(_C)]
    for bi in range(_BG):
        b = gb * _BG + bi
        for p in range(_P):
            ymin = boxes_ref[b, p, 0]
            xmin = boxes_ref[b, p, 1]
            ph = boxes_ref[b, p, 2]
            pw = boxes_ref[b, p, 3]
            ys = (jnp.minimum(ymin, _H - _WIN_Y + 5) // 8) * 8
            xs = (jnp.minimum(xmin, _W - _WIN_X + 1) // 128) * 128
            wyt = _weights_un(ph, ymin - ys, _WIN_Y, transposed=True)
            wx = _weights_un(pw, xmin - xs, _WIN_X, transposed=False)
            rty = 1.0 / jnp.sum(wyt, axis=1, keepdims=True)
            rtx = 1.0 / _dot(ones_x, wx)
            scale = rty * rtx
            ms = []
            for c in range(_C):
                crop = g_ref[bi, c, pl.ds(ys, _WIN_Y), pl.ds(xs, _WIN_X)]
                ms.append(_dot(wyt, crop))
            m_all = jnp.concatenate(ms, axis=0)
            o_all = _dot(m_all, wx)
            for c in range(_C):
                o = o_all[c * _OUT:(c + 1) * _OUT, :]
                accs[c] = accs[c] + o * scale
    for c in range(_C):
        out_ref[c, :, :] += accs[c]

    @pl.when(gb == (_B // _BG) - 1)
    def _finish():
        out_ref[...] = out_ref[...] * pg_ref[...]


@functools.partial(jax.jit, static_argnames=())
def kernel(gradients, patch_boxes, transform_decisions, patch_grads):
    del transform_decisions
    g = jnp.transpose(gradients, (0, 3, 1, 2))
    pg = jnp.transpose(patch_grads, (2, 0, 1))
    out = pl.pallas_call(
        _body,
        grid=(_B // _BG,),
        in_specs=[
            pl.BlockSpec((_BG, _C, _H, _W), lambda i: (i, 0, 0, 0)),
            pl.BlockSpec(memory_space=pltpu.SMEM),
            pl.BlockSpec((_C, _OUT, _OUT), lambda i: (0, 0, 0)),
        ],
        out_specs=pl.BlockSpec((_C, _OUT, _OUT), lambda i: (0, 0, 0)),
        out_shape=jax.ShapeDtypeStruct((_C, _OUT, _OUT), jnp.float32),
    )(g, patch_boxes, pg)
    return jnp.transpose(out, (1, 2, 0))


# speedup vs baseline: 1.2206x; 1.0996x over previous
import functools

import jax
import jax.numpy as jnp
from jax.experimental import pallas as pl
from jax.experimental.pallas import tpu as pltpu

_B, _H, _W, _C = 16, 512, 512, 3
_P = 8
_OUT = 64
_BG = 4
_WIN_Y = 136
_WIN_X = 256
_WX = 128


def _weights_un(length, off, win, *, transposed):
    lf = length.astype(jnp.float32)
    inv_scale = lf * (1.0 / _OUT)
    ks = jnp.maximum(inv_scale, 1.0)
    if transposed:
        shape = (_OUT, win)
        s_dim, i_dim = 0, 1
    else:
        shape = (win, _OUT)
        s_dim, i_dim = 1, 0
    s = jax.lax.broadcasted_iota(jnp.int32, shape, s_dim)
    i = jax.lax.broadcasted_iota(jnp.int32, shape, i_dim)
    sample = (s.astype(jnp.float32) + 0.5) * inv_scale - 0.5
    r = i.astype(jnp.float32) - off.astype(jnp.float32)
    x = jnp.abs(sample - r) * (1.0 / ks)
    w = jnp.maximum(0.0, 1.0 - x)
    return w * ((r >= 0.0) & (r < lf)).astype(jnp.float32)


def _dot(a, b):
    return jax.lax.dot_general(
        a, b, (((1,), (0,)), ((), ())),
        precision=jax.lax.Precision.DEFAULT,
        preferred_element_type=jnp.float32)


def _body(g_ref, boxes_ref, pg_ref, out_ref):
    gb = pl.program_id(0)

    @pl.when(gb == 0)
    def _init():
        out_ref[...] = jnp.zeros_like(out_ref)

    ones_x = jnp.ones((1, _WX), jnp.float32)
    for bi in range(_BG):
        b = gb * _BG + bi
        ms = []
        wxs = []
        for p in range(_P):
            ymin = boxes_ref[b, p, 0]
            xmin = boxes_ref[b, p, 1]
            ph = boxes_ref[b, p, 2]
            pw = boxes_ref[b, p, 3]
            ys = (jnp.minimum(ymin, _H - _WIN_Y + 5) // 8) * 8
            xs = (jnp.minimum(xmin, _W - _WIN_X + 1) // 128) * 128
            dx = xmin - xs
            wyt = _weights_un(ph, ymin - ys, _WIN_Y, transposed=True)
            wyt = wyt * (1.0 / jnp.sum(wyt, axis=1, keepdims=True))
            wx = _weights_un(pw, jnp.int32(0), _WX, transposed=False)
            wx = wx * (1.0 / _dot(ones_x, wx))
            wxs.append(wx)
            mc = []
            for c in range(_C):
                crop = g_ref[bi, c, pl.ds(ys, _WIN_Y), pl.ds(xs, _WIN_X)]
                mc.append(_dot(wyt, crop))
            m3 = jnp.concatenate(mc, axis=0)
            ms.append(pltpu.roll(m3, _WIN_X - dx, axis=1)[:, :_WX])
        m_img = jnp.concatenate(ms, axis=1)
        wx_img = jnp.concatenate(wxs, axis=0)
        o_img = _dot(m_img, wx_img)
        for c in range(_C):
            out_ref[c, :, :] += o_img[c * _OUT:(c + 1) * _OUT, :]

    @pl.when(gb == (_B // _BG) - 1)
    def _finish():
        out_ref[...] = out_ref[...] * pg_ref[...]


@functools.partial(jax.jit, static_argnames=())
def kernel(gradients, patch_boxes, transform_decisions, patch_grads):
    del transform_decisions
    g = jnp.transpose(gradients, (0, 3, 1, 2))
    pg = jnp.transpose(patch_grads, (2, 0, 1))
    out = pl.pallas_call(
        _body,
        grid=(_B // _BG,),
        in_specs=[
            pl.BlockSpec((_BG, _C, _H, _W), lambda i: (i, 0, 0, 0)),
            pl.BlockSpec(memory_space=pltpu.SMEM),
            pl.BlockSpec((_C, _OUT, _OUT), lambda i: (0, 0, 0)),
        ],
        out_specs=pl.BlockSpec((_C, _OUT, _OUT), lambda i: (0, 0, 0)),
        out_shape=jax.ShapeDtypeStruct((_C, _OUT, _OUT), jnp.float32),
    )(g, patch_boxes, pg)
    return jnp.transpose(out, (1, 2, 0))
